# baseline (speedup 1.0000x reference)
import jax
import jax.numpy as jnp
from jax import lax
from jax.experimental import pallas as pl
from jax.experimental.pallas import tpu as pltpu
from jax.experimental.pallas import tpu_sc as plsc

D = 128
NROWS = 100
N = 100000
NC, NS = 2, 16
NW = NC * NS

N1 = 48000
BPW = N1 // NW
CHUNK = 125
CPW = BPW // CHUNK
NBUF = 5
NLANE = 16
GRP = (CHUNK // NLANE) * NLANE
TGRP = CHUNK - NLANE
ISTAGE = (BPW // 8 + 2) * 8

N2 = N - N1
TR = 2000
T1 = N1 // TR
T2 = N2 // TR


def _emb_body(z_hbm, tab_hbm, out_hbm, tab_v, idx_v, stage, wsems):
    wid = lax.axis_index("s") * NC + lax.axis_index("c")
    base = wid * BPW
    astart = lax.min((base // 8) * 8, N - ISTAGE)
    s = base - astart
    pltpu.sync_copy(tab_hbm, tab_v)
    pltpu.sync_copy(z_hbm.at[pl.ds(astart, ISTAGE)], idx_v)

    CW = CHUNK * D

    def write(j):
        boff = (j % NBUF) * CW
        return pltpu.make_async_copy(
            stage.at[pl.ds(boff, CW)],
            out_hbm.at[pl.ds((base + j * CHUNK) * D, CW)],
            wsems.at[j % NBUF])

    def chunk(j, carry):
        boff = (j % NBUF) * CW

        @pl.when(j >= NBUF)
        def _():
            write(j - NBUF).wait()

        def group(i):
            zv = idx_v[pl.ds(s + j * CHUNK + i, NLANE)]
            for k in range(NLANE):
                off = zv[k] * D
                dst = boff + i * D + k * D
                for c in range(D // NLANE):
                    stage[pl.ds(dst + c * NLANE, NLANE)] = (
                        tab_v[pl.ds(off + c * NLANE, NLANE)])

        @plsc.parallel_loop(0, GRP + NLANE, step=NLANE)
        def _group(i):
            group(lax.min(i, TGRP))

        write(j).start()
        return carry

    lax.fori_loop(0, CPW, chunk, 0)
    for j in range(CPW - NBUF, CPW):
        write(j).wait()


def _tc_body(z_ref, t0_ref, t1_ref, t2_ref, y_ref, out_ref):
    del y_ref
    z = z_ref[...]
    cols = lax.broadcasted_iota(jnp.int32, (TR, NROWS), 1)
    onehot = (z == cols).astype(jnp.bfloat16)
    acc = jnp.dot(onehot, t0_ref[...], preferred_element_type=jnp.float32)
    acc += jnp.dot(onehot, t1_ref[...], preferred_element_type=jnp.float32)
    acc += jnp.dot(onehot, t2_ref[...], preferred_element_type=jnp.float32)
    out_ref[...] = acc


@jax.jit
def _emb(z1d, tab2d):
    sc = pl.kernel(
        _emb_body,
        out_type=jax.ShapeDtypeStruct((N * D,), jnp.float32),
        mesh=plsc.VectorSubcoreMesh(core_axis_name="c", subcore_axis_name="s"),
        scratch_types=[
            pltpu.VMEM((NROWS * D,), jnp.float32),
            pltpu.VMEM((ISTAGE,), jnp.int32),
            pltpu.VMEM((NBUF * CHUNK * D,), jnp.float32),
            pltpu.SemaphoreType.DMA((NBUF,)),
        ],
    )
    y = sc(z1d, tab2d.reshape(-1)).reshape(N, D)

    t0 = tab2d.astype(jnp.bfloat16)
    r1 = tab2d - t0.astype(jnp.float32)
    t1 = r1.astype(jnp.bfloat16)
    t2 = (r1 - t1.astype(jnp.float32)).astype(jnp.bfloat16)

    tab_spec = pl.BlockSpec((NROWS, D), lambda i: (0, 0))
    tc = pl.pallas_call(
        _tc_body,
        grid=(T2,),
        in_specs=[
            pl.BlockSpec((TR, 1), lambda i: (i, 0)),
            tab_spec, tab_spec, tab_spec,
            pl.BlockSpec(memory_space=pl.ANY),
        ],
        out_specs=pl.BlockSpec((TR, D), lambda i: (T1 + i, 0)),
        out_shape=jax.ShapeDtypeStruct((N, D), jnp.float32),
        input_output_aliases={4: 0},
    )
    return tc(z1d[N1:].reshape(N2, 1), t0, t1, t2, y)


def kernel(Z, emb_weight):
    return _emb(Z.astype(jnp.int32), emb_weight)


# speedup vs baseline: 1.1731x; 1.1731x over previous
import jax
import jax.numpy as jnp
from jax import lax
from jax.experimental import pallas as pl
from jax.experimental.pallas import tpu as pltpu
from jax.experimental.pallas import tpu_sc as plsc

D = 128
NROWS = 100
N = 100000
NC, NS = 2, 16
NW = NC * NS

N1 = 48000
BPW = N1 // NW
CHUNK = 125
CPW = BPW // CHUNK
NBUF = 5
NLANE = 16
GRP = (CHUNK // NLANE) * NLANE
TGRP = CHUNK - NLANE
ISTAGE = (BPW // 8 + 2) * 8

N2 = N - N1
TR = 4000
T1 = N1 // TR
T2 = N2 // TR


def _emb_body(z_hbm, tab_hbm, out_hbm, tab_v, idx_v, stage, wsems):
    wid = lax.axis_index("s") * NC + lax.axis_index("c")
    base = wid * BPW
    astart = lax.min((base // 8) * 8, N - ISTAGE)
    s = base - astart
    pltpu.sync_copy(tab_hbm, tab_v)
    pltpu.sync_copy(z_hbm.at[pl.ds(astart, ISTAGE)], idx_v)

    CW = CHUNK * D

    def write(j):
        boff = (j % NBUF) * CW
        return pltpu.make_async_copy(
            stage.at[pl.ds(boff, CW)],
            out_hbm.at[pl.ds((base + j * CHUNK) * D, CW)],
            wsems.at[j % NBUF])

    def chunk(j, carry):
        boff = (j % NBUF) * CW

        @pl.when(j >= NBUF)
        def _():
            write(j - NBUF).wait()

        def group(i):
            zv = idx_v[pl.ds(s + j * CHUNK + i, NLANE)]
            for k in range(NLANE):
                off = zv[k] * D
                dst = boff + i * D + k * D
                for c in range(D // NLANE):
                    stage[pl.ds(dst + c * NLANE, NLANE)] = (
                        tab_v[pl.ds(off + c * NLANE, NLANE)])

        @plsc.parallel_loop(0, GRP + NLANE, step=NLANE)
        def _group(i):
            group(lax.min(i, TGRP))

        write(j).start()
        return carry

    lax.fori_loop(0, CPW, chunk, 0)
    for j in range(CPW - NBUF, CPW):
        write(j).wait()


def _tc_body(z_ref, t0_ref, t1_ref, t2_ref, y_ref, out_ref):
    del y_ref
    z = z_ref[...]
    cols = lax.broadcasted_iota(jnp.int32, (TR, NROWS), 1)
    onehot = (z == cols).astype(jnp.float32)
    acc = jnp.dot(onehot, t0_ref[...], preferred_element_type=jnp.float32)
    acc += jnp.dot(onehot, t1_ref[...], preferred_element_type=jnp.float32)
    acc += jnp.dot(onehot, t2_ref[...], preferred_element_type=jnp.float32)
    out_ref[...] = acc


@jax.jit
def _emb(z1d, tab2d):
    sc = pl.kernel(
        _emb_body,
        out_type=jax.ShapeDtypeStruct((N * D,), jnp.float32),
        mesh=plsc.VectorSubcoreMesh(core_axis_name="c", subcore_axis_name="s"),
        scratch_types=[
            pltpu.VMEM((NROWS * D,), jnp.float32),
            pltpu.VMEM((ISTAGE,), jnp.int32),
            pltpu.VMEM((NBUF * CHUNK * D,), jnp.float32),
            pltpu.SemaphoreType.DMA((NBUF,)),
        ],
    )
    y = sc(z1d, tab2d.reshape(-1)).reshape(N, D)

    t0 = tab2d.astype(jnp.bfloat16).astype(jnp.float32)
    r1 = tab2d - t0
    t1 = r1.astype(jnp.bfloat16).astype(jnp.float32)
    t2 = (r1 - t1).astype(jnp.bfloat16).astype(jnp.float32)

    tab_spec = pl.BlockSpec((NROWS, D), lambda i: (0, 0))
    tc = pl.pallas_call(
        _tc_body,
        grid=(T2,),
        in_specs=[
            pl.BlockSpec((TR, 1), lambda i: (i, 0)),
            tab_spec, tab_spec, tab_spec,
            pl.BlockSpec(memory_space=pl.ANY),
        ],
        out_specs=pl.BlockSpec((TR, D), lambda i: (T1 + i, 0)),
        out_shape=jax.ShapeDtypeStruct((N, D), jnp.float32),
        input_output_aliases={4: 0},
    )
    return tc(z1d[N1:].reshape(N2, 1), t0, t1, t2, y)


def kernel(Z, emb_weight):
    return _emb(Z.astype(jnp.int32), emb_weight)


# speedup vs baseline: 1.9468x; 1.6596x over previous
import jax
import jax.numpy as jnp
from jax import lax
from jax.experimental import pallas as pl
from jax.experimental.pallas import tpu as pltpu
from jax.experimental.pallas import tpu_sc as plsc

D = 128
NROWS = 100
N = 100000
NC, NS = 2, 16
NW = NC * NS
BPW = N // NW
CHUNK = 125
CPW = BPW // CHUNK
NBUF = 5
NLANE = 16
GRP = (CHUNK // NLANE) * NLANE
TGRP = CHUNK - NLANE
ISTAGE = (BPW // 8 + 2) * 8


def _emb_body(z_hbm, tab_hbm, out_hbm, tab_v, idx_v, stage, wsems):
    wid = lax.axis_index("s") * NC + lax.axis_index("c")
    base = wid * BPW
    astart = lax.min((base // 8) * 8, N - ISTAGE)
    s = base - astart
    pltpu.sync_copy(tab_hbm, tab_v)
    pltpu.sync_copy(z_hbm.at[pl.ds(astart, ISTAGE)], idx_v)

    CW = CHUNK * D

    def write(j):
        boff = (j % NBUF) * CW
        return pltpu.make_async_copy(
            stage.at[pl.ds(boff, CW)],
            out_hbm.at[pl.ds((base + j * CHUNK) * D, CW)],
            wsems.at[j % NBUF])

    def chunk(j, carry):
        boff = (j % NBUF) * CW

        @pl.when(j >= NBUF)
        def _():
            write(j - NBUF).wait()

        def group(i):
            zv = idx_v[pl.ds(s + j * CHUNK + i, NLANE)]
            for k in range(NLANE):
                off = zv[k] * D
                dst = boff + i * D + k * D
                row = [tab_v[pl.ds(off + c * NLANE, NLANE)]
                       for c in range(D // NLANE)]
                for c in range(D // NLANE):
                    stage[pl.ds(dst + c * NLANE, NLANE)] = row[c]

        @plsc.parallel_loop(0, GRP + NLANE, step=NLANE)
        def _group(i):
            group(lax.min(i, TGRP))

        write(j).start()
        return carry

    lax.fori_loop(0, CPW, chunk, 0)
    for j in range(CPW - NBUF, CPW):
        write(j).wait()


@jax.jit
def _emb(z1d, tab_flat):
    f = pl.kernel(
        _emb_body,
        out_type=jax.ShapeDtypeStruct((N * D,), jnp.float32),
        mesh=plsc.VectorSubcoreMesh(core_axis_name="c", subcore_axis_name="s"),
        scratch_types=[
            pltpu.VMEM((NROWS * D,), jnp.float32),
            pltpu.VMEM((ISTAGE,), jnp.int32),
            pltpu.VMEM((NBUF * CHUNK * D,), jnp.float32),
            pltpu.SemaphoreType.DMA((NBUF,)),
        ],
    )
    return f(z1d, tab_flat)


def kernel(Z, emb_weight):
    out = _emb(Z.astype(jnp.int32), emb_weight.reshape(-1))
    return out.reshape(N, D)
